# baseline (speedup 1.0000x reference)
import functools

import jax
import jax.numpy as jnp
from jax import lax
from jax.experimental import pallas as pl
from jax.experimental.pallas import tpu as pltpu
from jax.experimental.pallas import tpu_sc as plsc

NUM_CLUSTERS = 1000000
Z_N = 16
BATCH = 16384
HIST = 200

_INFO = plsc.get_sparse_core_info()
_NC = _INFO.num_cores
_NS = _INFO.num_subcores
_NW = _NC * _NS

_BT = BATCH // _NW
_TC = _BT // 128
_HPAIRS = HIST // 2


def _body(x_hbm, mu_hbm, lv_hbm, out_z, out_mu, out_lv,
          idx0, idx1, mu0, mu1, lv0, lv1, tmu0, tmu1, tlv0, tlv1,
          si0, si1, sgm0, sgm1, sgl0, sgl1, swz0, swz1, swm0, swm1, swl0, swl1):
    idx = (idx0, idx1)
    mu_v = (mu0, mu1)
    lv_v = (lv0, lv1)
    tmu = (tmu0, tmu1)
    tlv = (tlv0, tlv1)
    si = (si0, si1)
    sgm = (sgm0, sgm1)
    sgl = (sgl0, sgl1)
    swz = (swz0, swz1)
    swm = (swm0, swm1)
    swl = (swl0, swl1)

    wid = lax.axis_index("s") * _NC + lax.axis_index("c")
    bt0 = wid * _TC
    iota = lax.iota(jnp.int32, 16)

    def start_idx(b, h):
        ht = h // 8
        hs = h % 8
        for j in range(_TC):
            pltpu.async_copy(x_hbm.at[ht, bt0 + j, hs, :],
                             idx[b].at[pl.ds(j * 128, 128)], si[b])

    def wait_idx(b):
        pltpu.make_async_copy(x_hbm.at[0, pl.ds(0, _TC), 0, :], idx[b], si[b]).wait()

    def start_gathers(b):
        pltpu.async_copy(mu_hbm.at[idx[b]], mu_v[b], sgm[b])
        pltpu.async_copy(lv_hbm.at[idx[b]], lv_v[b], sgl[b])

    def wait_gathers(b):
        pltpu.make_async_copy(mu_hbm.at[idx[b]], mu_v[b], sgm[b]).wait()
        pltpu.make_async_copy(lv_hbm.at[idx[b]], lv_v[b], sgl[b]).wait()

    def transpose(b):
        def _tblock(b16, carry):
            d0 = iota + b16 * 16
            l0 = (b16 % 8) * 16
            bt = b16 // 8
            for z in range(Z_N):
                d1 = jnp.full((16,), z, jnp.int32)
                tmu[b][z // 8, bt, z % 8, pl.ds(l0, 16)] = (
                    plsc.load_gather(mu_v[b], [d0, d1]))
            for z in range(Z_N):
                d1 = jnp.full((16,), z, jnp.int32)
                tlv[b][z // 8, bt, z % 8, pl.ds(l0, 16)] = (
                    plsc.load_gather(lv_v[b], [d0, d1]))
            return 0

        lax.fori_loop(0, _BT // 16, _tblock, 0)

    def start_writes(b, h):
        pltpu.async_copy(tmu[b], out_z.at[h, :, pl.ds(bt0, _TC), :, :], swz[b])
        pltpu.async_copy(tmu[b], out_mu.at[h, :, pl.ds(bt0, _TC), :, :], swm[b])
        pltpu.async_copy(tlv[b], out_lv.at[h, :, pl.ds(bt0, _TC), :, :], swl[b])

    def wait_writes(b):
        pltpu.make_async_copy(tmu[b], out_z.at[0, :, pl.ds(bt0, _TC), :, :], swz[b]).wait()
        pltpu.make_async_copy(tmu[b], out_mu.at[0, :, pl.ds(bt0, _TC), :, :], swm[b]).wait()
        pltpu.make_async_copy(tlv[b], out_lv.at[0, :, pl.ds(bt0, _TC), :, :], swl[b]).wait()

    start_idx(0, 0)
    start_idx(1, 1)
    wait_idx(0)
    start_gathers(0)

    def slot(b, h, first, last, prefetch):
        wait_gathers(b)
        if not last:
            wait_idx(1 - b)
            start_gathers(1 - b)
        if not first:
            wait_writes(b)
        if prefetch:
            start_idx(b, h + 2)
        transpose(b)
        start_writes(b, h)

    slot(0, 0, first=True, last=False, prefetch=True)
    slot(1, 1, first=True, last=False, prefetch=True)

    def pair_inner(j, _):
        h0 = 2 * j
        slot(0, h0, first=False, last=False, prefetch=True)
        slot(1, h0 + 1, first=False, last=False, prefetch=True)
        return 0

    lax.fori_loop(1, _HPAIRS - 1, pair_inner, 0)
    slot(0, 2 * (_HPAIRS - 1), first=False, last=False, prefetch=False)
    slot(1, 2 * (_HPAIRS - 1) + 1, first=False, last=True, prefetch=False)
    wait_writes(0)
    wait_writes(1)


@jax.jit
def _run(x5, embed_mu, embed_logvar):
    mesh = plsc.VectorSubcoreMesh(core_axis_name="c", subcore_axis_name="s")
    dma = pltpu.SemaphoreType.DMA
    return pl.kernel(
        _body,
        out_type=(
            jax.ShapeDtypeStruct((HIST, 2, 128, 8, 128), jnp.float32),
            jax.ShapeDtypeStruct((HIST, 2, 128, 8, 128), jnp.float32),
            jax.ShapeDtypeStruct((HIST, 2, 128, 8, 128), jnp.float32),
        ),
        mesh=mesh,
        scratch_types=(
            [pltpu.VMEM((_BT,), jnp.int32)] * 2
            + [pltpu.VMEM((_BT, Z_N + 1), jnp.float32)] * 4
            + [pltpu.VMEM((2, _TC, 8, 128), jnp.float32)] * 4
            + [dma] * 12
        ),
        compiler_params=pltpu.CompilerParams(
            use_tc_tiling_on_sc=False, needs_layout_passes=False),
    )(x5, embed_mu, embed_logvar)


def kernel(x, embed_mu, embed_logvar):
    x5 = x.T.reshape(25, 8, 128, 128).transpose(0, 2, 1, 3).astype(jnp.int32)
    mu_p = jnp.pad(embed_mu, ((0, 0), (0, 1)))
    lv_p = jnp.pad(embed_logvar, ((0, 0), (0, 1)))
    out_z, out_mu, out_lv = _run(x5, mu_p, lv_p)
    perm = (2, 4, 0, 1, 3)
    z = out_z.transpose(perm).reshape(BATCH, HIST, Z_N)
    mu = out_mu.transpose(perm).reshape(BATCH, HIST, Z_N)
    logvar = out_lv.transpose(perm).reshape(BATCH, HIST, Z_N)
    return (z, mu, logvar)


# speedup vs baseline: 1.7541x; 1.7541x over previous
import functools

import jax
import jax.numpy as jnp
from jax import lax
from jax.experimental import pallas as pl
from jax.experimental.pallas import tpu as pltpu
from jax.experimental.pallas import tpu_sc as plsc

NUM_CLUSTERS = 1000000
Z_N = 16
BATCH = 16384
HIST = 200

_INFO = plsc.get_sparse_core_info()
_NC = _INFO.num_cores
_NS = _INFO.num_subcores
_NW = _NC * _NS

_BT = BATCH // _NW
_TC = _BT // 128
_HPAIRS = HIST // 2


def _body(x_hbm, mu_hbm, lv_hbm, out_z, out_mu, out_lv,
          idx0, idx1, mu0, mu1, lv0, lv1, tmu0, tmu1, tlv0, tlv1,
          si0, si1, sgm0, sgm1, sgl0, sgl1, swz0, swz1, swm0, swm1, swl0, swl1):
    idx = (idx0, idx1)
    mu_v = (mu0, mu1)
    lv_v = (lv0, lv1)
    tmu = (tmu0, tmu1)
    tlv = (tlv0, tlv1)
    si = (si0, si1)
    sgm = (sgm0, sgm1)
    sgl = (sgl0, sgl1)
    swz = (swz0, swz1)
    swm = (swm0, swm1)
    swl = (swl0, swl1)

    wid = lax.axis_index("s") * _NC + lax.axis_index("c")
    bt0 = wid * _TC
    iota = lax.iota(jnp.int32, 16)

    def start_idx(b, h):
        ht = h // 8
        hs = h % 8
        for j in range(_TC):
            pltpu.async_copy(x_hbm.at[ht, bt0 + j, hs, :],
                             idx[b].at[pl.ds(j * 128, 128)], si[b])

    def wait_idx(b):
        pltpu.make_async_copy(x_hbm.at[0, pl.ds(0, _TC), 0, :], idx[b], si[b]).wait()

    def start_gathers(b):
        pltpu.async_copy(mu_hbm.at[idx[b]], mu_v[b], sgm[b])
        pltpu.async_copy(lv_hbm.at[idx[b]], lv_v[b], sgl[b])

    def wait_gathers(b):
        pltpu.make_async_copy(mu_hbm.at[idx[b]], mu_v[b], sgm[b]).wait()
        pltpu.make_async_copy(lv_hbm.at[idx[b]], lv_v[b], sgl[b]).wait()

    rotm = [(iota + s) % 16 for s in range(Z_N)]
    dzt = [r // 8 for r in rotm]
    dzs = [r % 8 for r in rotm]

    def transpose(b):
        def _tblock(b16, carry):
            d0 = iota + b16 * 16
            d3 = iota + (b16 % 8) * 16
            da = jnp.full((16,), b16 // 8, jnp.int32)
            for s in range(Z_N):
                v = plsc.load_gather(mu_v[b], [d0, rotm[s]])
                plsc.store_scatter(tmu[b], [dzt[s], da, dzs[s], d3], v)
            for s in range(Z_N):
                v = plsc.load_gather(lv_v[b], [d0, rotm[s]])
                plsc.store_scatter(tlv[b], [dzt[s], da, dzs[s], d3], v)
            return 0

        lax.fori_loop(0, _BT // 16, _tblock, 0)

    def start_writes(b, h):
        pltpu.async_copy(tmu[b], out_z.at[h, :, pl.ds(bt0, _TC), :, :], swz[b])
        pltpu.async_copy(tmu[b], out_mu.at[h, :, pl.ds(bt0, _TC), :, :], swm[b])
        pltpu.async_copy(tlv[b], out_lv.at[h, :, pl.ds(bt0, _TC), :, :], swl[b])

    def wait_writes(b):
        pltpu.make_async_copy(tmu[b], out_z.at[0, :, pl.ds(bt0, _TC), :, :], swz[b]).wait()
        pltpu.make_async_copy(tmu[b], out_mu.at[0, :, pl.ds(bt0, _TC), :, :], swm[b]).wait()
        pltpu.make_async_copy(tlv[b], out_lv.at[0, :, pl.ds(bt0, _TC), :, :], swl[b]).wait()

    start_idx(0, 0)
    start_idx(1, 1)
    wait_idx(0)
    start_gathers(0)

    def slot(b, h, first, last, prefetch):
        wait_gathers(b)
        if not last:
            wait_idx(1 - b)
            start_gathers(1 - b)
        if not first:
            wait_writes(b)
        if prefetch:
            start_idx(b, h + 2)
        transpose(b)
        start_writes(b, h)

    slot(0, 0, first=True, last=False, prefetch=True)
    slot(1, 1, first=True, last=False, prefetch=True)

    def pair_inner(j, _):
        h0 = 2 * j
        slot(0, h0, first=False, last=False, prefetch=True)
        slot(1, h0 + 1, first=False, last=False, prefetch=True)
        return 0

    lax.fori_loop(1, _HPAIRS - 1, pair_inner, 0)
    slot(0, 2 * (_HPAIRS - 1), first=False, last=False, prefetch=False)
    slot(1, 2 * (_HPAIRS - 1) + 1, first=False, last=True, prefetch=False)
    wait_writes(0)
    wait_writes(1)


@jax.jit
def _run(x5, embed_mu, embed_logvar):
    mesh = plsc.VectorSubcoreMesh(core_axis_name="c", subcore_axis_name="s")
    dma = pltpu.SemaphoreType.DMA
    return pl.kernel(
        _body,
        out_type=(
            jax.ShapeDtypeStruct((HIST, 2, 128, 8, 128), jnp.float32),
            jax.ShapeDtypeStruct((HIST, 2, 128, 8, 128), jnp.float32),
            jax.ShapeDtypeStruct((HIST, 2, 128, 8, 128), jnp.float32),
        ),
        mesh=mesh,
        scratch_types=(
            [pltpu.VMEM((_BT,), jnp.int32)] * 2
            + [pltpu.VMEM((_BT, Z_N), jnp.float32)] * 4
            + [pltpu.VMEM((2, _TC, 8, 128), jnp.float32)] * 4
            + [dma] * 12
        ),
        compiler_params=pltpu.CompilerParams(
            use_tc_tiling_on_sc=False, needs_layout_passes=False),
    )(x5, embed_mu, embed_logvar)


def kernel(x, embed_mu, embed_logvar):
    x5 = x.T.reshape(25, 8, 128, 128).transpose(0, 2, 1, 3).astype(jnp.int32)
    out_z, out_mu, out_lv = _run(x5, embed_mu, embed_logvar)
    perm = (2, 4, 0, 1, 3)
    z = out_z.transpose(perm).reshape(BATCH, HIST, Z_N)
    mu = out_mu.transpose(perm).reshape(BATCH, HIST, Z_N)
    logvar = out_lv.transpose(perm).reshape(BATCH, HIST, Z_N)
    return (z, mu, logvar)
